# baseline (speedup 1.0000x reference)
import functools

import jax
import jax.numpy as jnp
from jax import lax
from jax.experimental import pallas as pl
from jax.experimental.pallas import tpu as pltpu
from jax.experimental.pallas import tpu_sc as plsc

N = 10000
D = 128
E = 320000
OUTP = 16

NC = 2
NS = 16
NW = NC * NS
NP = 10240
RPT = NP // NS
CHUNK = 128
EPWP = 10240
EP = EPWP * NW
SEG = 4
NCHUNK = EPWP // (SEG * CHUNK)
NBUF = 2

_mesh = plsc.VectorSubcoreMesh(
    core_axis_name="c", subcore_axis_name="s", num_cores=NC, num_subcores=NS
)


@functools.partial(
    pl.kernel,
    out_type=jax.ShapeDtypeStruct((NC, NP, D), jnp.float32),
    mesh=_mesh,
    scratch_types=[
        pltpu.VMEM((NCHUNK, CHUNK), jnp.int32),
        pltpu.VMEM((CHUNK, D), jnp.float32),
        pltpu.VMEM_SHARED((NP, D), jnp.float32),
    ],
)
def _sc_deg(edges_r, zeros, ones, out, dst_v, ones_v, acc):
    c = lax.axis_index("c")
    s = lax.axis_index("s")
    w = c * NS + s
    pltpu.sync_copy(ones.at[:], ones_v)
    pltpu.sync_copy(zeros.at[pl.ds(s * RPT, RPT)], acc.at[pl.ds(s * RPT, RPT)])
    plsc.subcore_barrier()

    for seg in range(SEG):
        pltpu.sync_copy(edges_r.at[1, w, seg], dst_v)

        def step(j, carry):
            pltpu.sync_copy(ones_v, acc.at[dst_v.at[j]], add=True)
            return carry

        lax.fori_loop(0, NCHUNK, step, 0)
    plsc.subcore_barrier()
    pltpu.sync_copy(acc.at[pl.ds(s * RPT, RPT)], out.at[c, pl.ds(s * RPT, RPT)])


def _make_sc_scatter(d):
    @functools.partial(
        pl.kernel,
        out_type=jax.ShapeDtypeStruct((NC, NP, d), jnp.float32),
        mesh=_mesh,
        scratch_types=(
            [
                pltpu.VMEM((NCHUNK, CHUNK), jnp.int32),
                pltpu.VMEM((NCHUNK, CHUNK), jnp.int32),
            ]
            + [pltpu.VMEM((CHUNK, d), jnp.float32)] * NBUF
            + [pltpu.SemaphoreType.DMA] * (2 * NBUF)
            + [pltpu.VMEM_SHARED((NP, d), jnp.float32)]
        ),
    )
    def sc_scatter(ht, edges_r, zeros, out, src_v, dst_v,
                   r0, r1, g0, g1, s0, s1, acc):
        rows = (r0, r1)
        gsem = (g0, g1)
        ssem = (s0, s1)
        c = lax.axis_index("c")
        s = lax.axis_index("s")
        w = c * NS + s
        pltpu.sync_copy(zeros.at[pl.ds(s * RPT, RPT)], acc.at[pl.ds(s * RPT, RPT)])
        plsc.subcore_barrier()

        for seg in range(SEG):
            pltpu.sync_copy(edges_r.at[0, w, seg], src_v)
            pltpu.sync_copy(edges_r.at[1, w, seg], dst_v)

            for b in range(NBUF):
                pltpu.async_copy(ht.at[src_v.at[b]], rows[b], gsem[b])

            def group(g, carry):
                j0 = g * NBUF
                for b in range(NBUF):
                    j = j0 + b
                    pltpu.make_async_copy(
                        ht.at[src_v.at[j]], rows[b], gsem[b]).wait()
                    pltpu.async_copy(rows[b], acc.at[dst_v.at[j]], ssem[b],
                                     add=True)
                    pltpu.make_async_copy(
                        rows[b], acc.at[dst_v.at[j]], ssem[b]).wait()

                    @pl.when(j + NBUF < NCHUNK)
                    def _():
                        pltpu.async_copy(
                            ht.at[src_v.at[j + NBUF]], rows[b], gsem[b])
                return carry

            lax.fori_loop(0, NCHUNK // NBUF, group, 0)
        plsc.subcore_barrier()
        pltpu.sync_copy(acc.at[pl.ds(s * RPT, RPT)], out.at[c, pl.ds(s * RPT, RPT)])

    return sc_scatter


_sc_scatter_full = _make_sc_scatter(D)


_R = 2000
_G = N // _R


def _prep_body(degp_ref, x_ref, w_ref, dinv_ref, ht_ref):
    dv = lax.rsqrt(1.0 + degp_ref[0][:, 0:1] + degp_ref[1][:, 0:1])
    dinv_ref[...] = dv
    ht_ref[...] = dv * jnp.dot(x_ref[...], w_ref[...],
                               preferred_element_type=jnp.float32)


def _tc_prep(degp, x, w0):
    return pl.pallas_call(
        _prep_body,
        grid=(_G,),
        in_specs=[
            pl.BlockSpec((2, _R, D), lambda i: (0, i, 0)),
            pl.BlockSpec((_R, D), lambda i: (i, 0)),
            pl.BlockSpec((D, D), lambda i: (0, 0)),
        ],
        out_specs=[
            pl.BlockSpec((_R, 1), lambda i: (i, 0)),
            pl.BlockSpec((_R, D), lambda i: (i, 0)),
        ],
        out_shape=[
            jax.ShapeDtypeStruct((N, 1), jnp.float32),
            jax.ShapeDtypeStruct((N, D), jnp.float32),
        ],
    )(degp, x, w0)


def _mid_body(p_ref, ht_ref, dinv_ref, b_ref, w_ref, o_ref):
    dv = dinv_ref[...]
    a = dv * (p_ref[0] + p_ref[1] + ht_ref[...]) + b_ref[...]
    a = jnp.where(a >= 0.0, a, 0.2 * a)
    o_ref[...] = dv * jnp.dot(a, w_ref[...], preferred_element_type=jnp.float32)


def _tc_mid(p, ht, dinv, b, w):
    dn = w.shape[1]
    return pl.pallas_call(
        _mid_body,
        grid=(_G,),
        in_specs=[
            pl.BlockSpec((2, _R, D), lambda i: (0, i, 0)),
            pl.BlockSpec((_R, D), lambda i: (i, 0)),
            pl.BlockSpec((_R, 1), lambda i: (i, 0)),
            pl.BlockSpec((1, D), lambda i: (0, 0)),
            pl.BlockSpec((D, dn), lambda i: (0, 0)),
        ],
        out_specs=pl.BlockSpec((_R, dn), lambda i: (i, 0)),
        out_shape=jax.ShapeDtypeStruct((N, dn), jnp.float32),
    )(p, ht, dinv, b, w)


def _g_body(p_ref, ht_ref, dinv_ref, b_ref, o_ref):
    dv = dinv_ref[...]
    a = dv * (p_ref[0] + p_ref[1] + ht_ref[...]) + b_ref[...]
    o_ref[...] = dv * jnp.where(a >= 0.0, a, 0.2 * a)


def _tc_g(p, ht, dinv, b):
    return pl.pallas_call(
        _g_body,
        grid=(_G,),
        in_specs=[
            pl.BlockSpec((2, _R, D), lambda i: (0, i, 0)),
            pl.BlockSpec((_R, D), lambda i: (i, 0)),
            pl.BlockSpec((_R, 1), lambda i: (i, 0)),
            pl.BlockSpec((1, D), lambda i: (0, 0)),
        ],
        out_specs=pl.BlockSpec((_R, D), lambda i: (i, 0)),
        out_shape=jax.ShapeDtypeStruct((N, D), jnp.float32),
    )(p, ht, dinv, b)


def _fin_body(p_ref, g_ref, dinv_ref, b_ref, w_ref, o_ref):
    dv = dinv_ref[...]
    a = p_ref[0] + p_ref[1] + g_ref[...]
    o_ref[...] = dv * jnp.dot(a, w_ref[...],
                              preferred_element_type=jnp.float32) + b_ref[...]


def _tc_fin(p, g, dinv, b, w):
    return pl.pallas_call(
        _fin_body,
        grid=(_G,),
        in_specs=[
            pl.BlockSpec((2, _R, D), lambda i: (0, i, 0)),
            pl.BlockSpec((_R, D), lambda i: (i, 0)),
            pl.BlockSpec((_R, 1), lambda i: (i, 0)),
            pl.BlockSpec((1, OUTP), lambda i: (0, 0)),
            pl.BlockSpec((D, OUTP), lambda i: (0, 0)),
        ],
        out_specs=pl.BlockSpec((_R, OUTP), lambda i: (i, 0)),
        out_shape=jax.ShapeDtypeStruct((N, OUTP), jnp.float32),
    )(p, g, dinv, b, w)


@jax.jit
def kernel(x, edge_index, W0, b0, W1, b1, W2, b2, W3, b3):
    pad = jnp.broadcast_to(
        jnp.array([[0], [NP - 1]], jnp.int32), (2, EP - E)
    )
    edges_r = jnp.concatenate([edge_index, pad], axis=1).reshape(
        2, NW, SEG, NCHUNK, CHUNK
    )
    zeros_full = jnp.zeros((NP, D), jnp.float32)
    ones = jnp.ones((CHUNK, D), jnp.float32)
    w3p = jnp.zeros((D, OUTP), jnp.float32).at[:, : W3.shape[1]].set(W3)
    b3p = jnp.zeros((1, OUTP), jnp.float32).at[0, : b3.shape[0]].set(b3)

    degp = _sc_deg(edges_r, zeros_full, ones)
    dinv, ht = _tc_prep(degp, x, W0)

    for b, w in ((b0, W1), (b1, W2)):
        p = _sc_scatter_full(ht, edges_r, zeros_full)
        ht = _tc_mid(p, ht, dinv, b.reshape(1, D), w)

    p = _sc_scatter_full(ht, edges_r, zeros_full)
    g = _tc_g(p, ht, dinv, b2.reshape(1, D))

    p = _sc_scatter_full(g, edges_r, zeros_full)
    out = _tc_fin(p, g, dinv, b3p, w3p)
    return out[:, :4]


# speedup vs baseline: 1.0610x; 1.0610x over previous
import functools

import jax
import jax.numpy as jnp
from jax import lax
from jax.experimental import pallas as pl
from jax.experimental.pallas import tpu as pltpu
from jax.experimental.pallas import tpu_sc as plsc

N = 10000
D = 128
E = 320000
OUTP = 16

NC = 2
NS = 16
NW = NC * NS
NP = 10240
RPT = NP // NS
CHUNK = 64
EPWP = 10240
EP = EPWP * NW
SEG = 4
NCHUNK = EPWP // (SEG * CHUNK)
NBUF = 4

_mesh = plsc.VectorSubcoreMesh(
    core_axis_name="c", subcore_axis_name="s", num_cores=NC, num_subcores=NS
)


@functools.partial(
    pl.kernel,
    out_type=jax.ShapeDtypeStruct((NC, NP, D), jnp.float32),
    mesh=_mesh,
    scratch_types=[
        pltpu.VMEM((NCHUNK, CHUNK), jnp.int32),
        pltpu.VMEM((CHUNK, D), jnp.float32),
        pltpu.VMEM_SHARED((NP, D), jnp.float32),
    ],
)
def _sc_deg(edges_r, zeros, ones, out, dst_v, ones_v, acc):
    c = lax.axis_index("c")
    s = lax.axis_index("s")
    w = c * NS + s
    pltpu.sync_copy(ones.at[:], ones_v)
    pltpu.sync_copy(zeros.at[pl.ds(s * RPT, RPT)], acc.at[pl.ds(s * RPT, RPT)])
    plsc.subcore_barrier()

    for seg in range(SEG):
        pltpu.sync_copy(edges_r.at[1, w, seg], dst_v)

        def step(j, carry):
            pltpu.sync_copy(ones_v, acc.at[dst_v.at[j]], add=True)
            return carry

        lax.fori_loop(0, NCHUNK, step, 0)
    plsc.subcore_barrier()
    pltpu.sync_copy(acc.at[pl.ds(s * RPT, RPT)], out.at[c, pl.ds(s * RPT, RPT)])


def _make_sc_scatter(d):
    @functools.partial(
        pl.kernel,
        out_type=jax.ShapeDtypeStruct((NC, NP, d), jnp.float32),
        mesh=_mesh,
        scratch_types=(
            [
                pltpu.VMEM((NCHUNK, CHUNK), jnp.int32),
                pltpu.VMEM((NCHUNK, CHUNK), jnp.int32),
            ]
            + [pltpu.VMEM((CHUNK, d), jnp.float32)] * NBUF
            + [pltpu.SemaphoreType.DMA] * (2 * NBUF)
            + [pltpu.VMEM_SHARED((NP, d), jnp.float32)]
        ),
    )
    def sc_scatter(ht, edges_r, zeros, out, src_v, dst_v,
                   r0, r1, r2, r3, g0, g1, g2, g3, s0, s1, s2, s3, acc):
        rows = (r0, r1, r2, r3)
        gsem = (g0, g1, g2, g3)
        ssem = (s0, s1, s2, s3)
        c = lax.axis_index("c")
        s = lax.axis_index("s")
        w = c * NS + s
        pltpu.sync_copy(zeros.at[pl.ds(s * RPT, RPT)], acc.at[pl.ds(s * RPT, RPT)])
        plsc.subcore_barrier()

        for seg in range(SEG):
            pltpu.sync_copy(edges_r.at[0, w, seg], src_v)
            pltpu.sync_copy(edges_r.at[1, w, seg], dst_v)

            for b in range(NBUF):
                pltpu.async_copy(ht.at[src_v.at[b]], rows[b], gsem[b])

            def group(g, carry):
                j0 = g * NBUF
                for b in range(NBUF):
                    j = j0 + b
                    pltpu.make_async_copy(
                        ht.at[src_v.at[j]], rows[b], gsem[b]).wait()
                    pltpu.async_copy(rows[b], acc.at[dst_v.at[j]], ssem[b],
                                     add=True)
                    pltpu.make_async_copy(
                        rows[b], acc.at[dst_v.at[j]], ssem[b]).wait()

                    @pl.when(j + NBUF < NCHUNK)
                    def _():
                        pltpu.async_copy(
                            ht.at[src_v.at[j + NBUF]], rows[b], gsem[b])
                return carry

            lax.fori_loop(0, NCHUNK // NBUF, group, 0)
        plsc.subcore_barrier()
        pltpu.sync_copy(acc.at[pl.ds(s * RPT, RPT)], out.at[c, pl.ds(s * RPT, RPT)])

    return sc_scatter


_sc_scatter_full = _make_sc_scatter(D)


_R = 2000
_G = N // _R


def _prep_body(degp_ref, x_ref, w_ref, dinv_ref, ht_ref):
    dv = lax.rsqrt(1.0 + degp_ref[0][:, 0:1] + degp_ref[1][:, 0:1])
    dinv_ref[...] = dv
    ht_ref[...] = dv * jnp.dot(x_ref[...], w_ref[...],
                               preferred_element_type=jnp.float32)


def _tc_prep(degp, x, w0):
    return pl.pallas_call(
        _prep_body,
        grid=(_G,),
        in_specs=[
            pl.BlockSpec((2, _R, D), lambda i: (0, i, 0)),
            pl.BlockSpec((_R, D), lambda i: (i, 0)),
            pl.BlockSpec((D, D), lambda i: (0, 0)),
        ],
        out_specs=[
            pl.BlockSpec((_R, 1), lambda i: (i, 0)),
            pl.BlockSpec((_R, D), lambda i: (i, 0)),
        ],
        out_shape=[
            jax.ShapeDtypeStruct((N, 1), jnp.float32),
            jax.ShapeDtypeStruct((N, D), jnp.float32),
        ],
    )(degp, x, w0)


def _mid_body(p_ref, ht_ref, dinv_ref, b_ref, w_ref, o_ref):
    dv = dinv_ref[...]
    a = dv * (p_ref[0] + p_ref[1] + ht_ref[...]) + b_ref[...]
    a = jnp.where(a >= 0.0, a, 0.2 * a)
    o_ref[...] = dv * jnp.dot(a, w_ref[...], preferred_element_type=jnp.float32)


def _tc_mid(p, ht, dinv, b, w):
    dn = w.shape[1]
    return pl.pallas_call(
        _mid_body,
        grid=(_G,),
        in_specs=[
            pl.BlockSpec((2, _R, D), lambda i: (0, i, 0)),
            pl.BlockSpec((_R, D), lambda i: (i, 0)),
            pl.BlockSpec((_R, 1), lambda i: (i, 0)),
            pl.BlockSpec((1, D), lambda i: (0, 0)),
            pl.BlockSpec((D, dn), lambda i: (0, 0)),
        ],
        out_specs=pl.BlockSpec((_R, dn), lambda i: (i, 0)),
        out_shape=jax.ShapeDtypeStruct((N, dn), jnp.float32),
    )(p, ht, dinv, b, w)


def _g_body(p_ref, ht_ref, dinv_ref, b_ref, o_ref):
    dv = dinv_ref[...]
    a = dv * (p_ref[0] + p_ref[1] + ht_ref[...]) + b_ref[...]
    o_ref[...] = dv * jnp.where(a >= 0.0, a, 0.2 * a)


def _tc_g(p, ht, dinv, b):
    return pl.pallas_call(
        _g_body,
        grid=(_G,),
        in_specs=[
            pl.BlockSpec((2, _R, D), lambda i: (0, i, 0)),
            pl.BlockSpec((_R, D), lambda i: (i, 0)),
            pl.BlockSpec((_R, 1), lambda i: (i, 0)),
            pl.BlockSpec((1, D), lambda i: (0, 0)),
        ],
        out_specs=pl.BlockSpec((_R, D), lambda i: (i, 0)),
        out_shape=jax.ShapeDtypeStruct((N, D), jnp.float32),
    )(p, ht, dinv, b)


def _fin_body(p_ref, g_ref, dinv_ref, b_ref, w_ref, o_ref):
    dv = dinv_ref[...]
    a = p_ref[0] + p_ref[1] + g_ref[...]
    o_ref[...] = dv * jnp.dot(a, w_ref[...],
                              preferred_element_type=jnp.float32) + b_ref[...]


def _tc_fin(p, g, dinv, b, w):
    return pl.pallas_call(
        _fin_body,
        grid=(_G,),
        in_specs=[
            pl.BlockSpec((2, _R, D), lambda i: (0, i, 0)),
            pl.BlockSpec((_R, D), lambda i: (i, 0)),
            pl.BlockSpec((_R, 1), lambda i: (i, 0)),
            pl.BlockSpec((1, OUTP), lambda i: (0, 0)),
            pl.BlockSpec((D, OUTP), lambda i: (0, 0)),
        ],
        out_specs=pl.BlockSpec((_R, OUTP), lambda i: (i, 0)),
        out_shape=jax.ShapeDtypeStruct((N, OUTP), jnp.float32),
    )(p, g, dinv, b, w)


@jax.jit
def kernel(x, edge_index, W0, b0, W1, b1, W2, b2, W3, b3):
    pad = jnp.broadcast_to(
        jnp.array([[0], [NP - 1]], jnp.int32), (2, EP - E)
    )
    edges_r = jnp.concatenate([edge_index, pad], axis=1).reshape(
        2, NW, SEG, NCHUNK, CHUNK
    )
    zeros_full = jnp.zeros((NP, D), jnp.float32)
    ones = jnp.ones((CHUNK, D), jnp.float32)
    w3p = jnp.zeros((D, OUTP), jnp.float32).at[:, : W3.shape[1]].set(W3)
    b3p = jnp.zeros((1, OUTP), jnp.float32).at[0, : b3.shape[0]].set(b3)

    degp = _sc_deg(edges_r, zeros_full, ones)
    dinv, ht = _tc_prep(degp, x, W0)

    for b, w in ((b0, W1), (b1, W2)):
        p = _sc_scatter_full(ht, edges_r, zeros_full)
        ht = _tc_mid(p, ht, dinv, b.reshape(1, D), w)

    p = _sc_scatter_full(ht, edges_r, zeros_full)
    g = _tc_g(p, ht, dinv, b2.reshape(1, D))

    p = _sc_scatter_full(g, edges_r, zeros_full)
    out = _tc_fin(p, g, dinv, b3p, w3p)
    return out[:, :4]


# speedup vs baseline: 3.3424x; 3.1502x over previous
import functools

import jax
import jax.numpy as jnp
from jax import lax
from jax.experimental import pallas as pl
from jax.experimental.pallas import tpu as pltpu
from jax.experimental.pallas import tpu_sc as plsc

N = 10000
D = 128
E = 320000
OUTP = 16

NC = 2
NS = 16
NW = NC * NS
NP = 10240
RPT = NP // NS
CHUNK = 64
EPWP = 10240
EP = EPWP * NW
SEG = 4
NCHUNK = EPWP // (SEG * CHUNK)
NBUF = 4

_mesh = plsc.VectorSubcoreMesh(
    core_axis_name="c", subcore_axis_name="s", num_cores=NC, num_subcores=NS
)


@functools.partial(
    pl.kernel,
    out_type=jax.ShapeDtypeStruct((NC, NP, D), jnp.float32),
    mesh=_mesh,
    scratch_types=[
        pltpu.VMEM((NCHUNK, CHUNK), jnp.int32),
        pltpu.VMEM((CHUNK, D), jnp.float32),
        pltpu.VMEM_SHARED((NP, D), jnp.float32),
    ],
)
def _sc_deg(edges_r, zeros, ones, out, dst_v, ones_v, acc):
    c = lax.axis_index("c")
    s = lax.axis_index("s")
    w = c * NS + s
    pltpu.sync_copy(ones.at[:], ones_v)
    pltpu.sync_copy(zeros.at[pl.ds(s * RPT, RPT)], acc.at[pl.ds(s * RPT, RPT)])
    plsc.subcore_barrier()

    for seg in range(SEG):
        pltpu.sync_copy(edges_r.at[1, w, seg], dst_v)

        def step(j, carry):
            pltpu.sync_copy(ones_v, acc.at[dst_v.at[j]], add=True)
            return carry

        lax.fori_loop(0, NCHUNK, step, 0)
    plsc.subcore_barrier()
    pltpu.sync_copy(acc.at[pl.ds(s * RPT, RPT)], out.at[c, pl.ds(s * RPT, RPT)])


def _make_sc_scatter(d):
    @functools.partial(
        pl.kernel,
        out_type=jax.ShapeDtypeStruct((NC, NP, d), jnp.float32),
        mesh=_mesh,
        scratch_types=(
            [
                pltpu.VMEM((NCHUNK, CHUNK), jnp.int32),
                pltpu.VMEM((NCHUNK, CHUNK), jnp.int32),
            ]
            + [pltpu.VMEM((CHUNK, d), jnp.float32)] * NBUF
            + [pltpu.SemaphoreType.DMA] * (2 * NBUF)
            + [pltpu.VMEM_SHARED((NP, d), jnp.float32)]
        ),
    )
    def sc_scatter(ht, edges_r, zeros, out, src_v, dst_v,
                   r0, r1, r2, r3, g0, g1, g2, g3, s0, s1, s2, s3, acc):
        rows = (r0, r1, r2, r3)
        gsem = (g0, g1, g2, g3)
        ssem = (s0, s1, s2, s3)
        c = lax.axis_index("c")
        s = lax.axis_index("s")
        w = c * NS + s
        pltpu.sync_copy(zeros.at[pl.ds(s * RPT, RPT)], acc.at[pl.ds(s * RPT, RPT)])
        plsc.subcore_barrier()

        for seg in range(SEG):
            pltpu.sync_copy(edges_r.at[0, w, seg], src_v)
            pltpu.sync_copy(edges_r.at[1, w, seg], dst_v)

            for b in range(NBUF):
                pltpu.async_copy(ht.at[src_v.at[b]], rows[b], gsem[b])

            def group(g, carry):
                j0 = g * NBUF
                for b in range(NBUF):
                    j = j0 + b
                    pltpu.make_async_copy(
                        ht.at[src_v.at[j]], rows[b], gsem[b]).wait()
                    pltpu.async_copy(rows[b], acc.at[dst_v.at[j]], ssem[b],
                                     add=True)
                    pltpu.make_async_copy(
                        rows[b], acc.at[dst_v.at[j]], ssem[b]).wait()

                    @pl.when(j + NBUF < NCHUNK)
                    def _():
                        pltpu.async_copy(
                            ht.at[src_v.at[j + NBUF]], rows[b], gsem[b])
                return carry

            lax.fori_loop(0, NCHUNK // NBUF, group, 0)
        plsc.subcore_barrier()
        pltpu.sync_copy(acc.at[pl.ds(s * RPT, RPT)], out.at[c, pl.ds(s * RPT, RPT)])

    return sc_scatter


_sc_scatter_full = _make_sc_scatter(D)


_R = 2000
_G = N // _R


def _prep_body(degp_ref, x_ref, w_ref, dinv_ref, ht_ref):
    dv = lax.rsqrt(1.0 + degp_ref[0][:, 0:1] + degp_ref[1][:, 0:1])
    dinv_ref[...] = dv
    ht_ref[...] = dv * jnp.dot(x_ref[...], w_ref[...],
                               preferred_element_type=jnp.float32)


def _tc_prep(degp, x, w0):
    return pl.pallas_call(
        _prep_body,
        grid=(_G,),
        in_specs=[
            pl.BlockSpec((2, _R, D), lambda i: (0, i, 0)),
            pl.BlockSpec((_R, D), lambda i: (i, 0)),
            pl.BlockSpec((D, D), lambda i: (0, 0)),
        ],
        out_specs=[
            pl.BlockSpec((_R, 1), lambda i: (i, 0)),
            pl.BlockSpec((_R, D), lambda i: (i, 0)),
        ],
        out_shape=[
            jax.ShapeDtypeStruct((N, 1), jnp.float32),
            jax.ShapeDtypeStruct((N, D), jnp.float32),
        ],
    )(degp, x, w0)


def _mid_body(p_ref, ht_ref, dinv_ref, b_ref, w_ref, o_ref):
    dv = dinv_ref[...]
    a = dv * (p_ref[0] + p_ref[1] + ht_ref[...]) + b_ref[...]
    a = jnp.where(a >= 0.0, a, 0.2 * a)
    o_ref[...] = dv * jnp.dot(a, w_ref[...], preferred_element_type=jnp.float32)


def _tc_mid(p, ht, dinv, b, w):
    dn = w.shape[1]
    return pl.pallas_call(
        _mid_body,
        grid=(_G,),
        in_specs=[
            pl.BlockSpec((2, _R, D), lambda i: (0, i, 0)),
            pl.BlockSpec((_R, D), lambda i: (i, 0)),
            pl.BlockSpec((_R, 1), lambda i: (i, 0)),
            pl.BlockSpec((1, D), lambda i: (0, 0)),
            pl.BlockSpec((D, dn), lambda i: (0, 0)),
        ],
        out_specs=pl.BlockSpec((_R, dn), lambda i: (i, 0)),
        out_shape=jax.ShapeDtypeStruct((N, dn), jnp.float32),
    )(p, ht, dinv, b, w)


def _g_body(p_ref, ht_ref, dinv_ref, b_ref, o_ref):
    dv = dinv_ref[...]
    a = dv * (p_ref[0] + p_ref[1] + ht_ref[...]) + b_ref[...]
    o_ref[...] = dv * jnp.where(a >= 0.0, a, 0.2 * a)


def _tc_g(p, ht, dinv, b):
    return pl.pallas_call(
        _g_body,
        grid=(_G,),
        in_specs=[
            pl.BlockSpec((2, _R, D), lambda i: (0, i, 0)),
            pl.BlockSpec((_R, D), lambda i: (i, 0)),
            pl.BlockSpec((_R, 1), lambda i: (i, 0)),
            pl.BlockSpec((1, D), lambda i: (0, 0)),
        ],
        out_specs=pl.BlockSpec((_R, D), lambda i: (i, 0)),
        out_shape=jax.ShapeDtypeStruct((N, D), jnp.float32),
    )(p, ht, dinv, b)


def _fin_body(p_ref, g_ref, dinv_ref, b_ref, w_ref, o_ref):
    dv = dinv_ref[...]
    a = p_ref[0] + p_ref[1] + g_ref[...]
    o_ref[...] = dv * jnp.dot(a, w_ref[...],
                              preferred_element_type=jnp.float32) + b_ref[...]


def _tc_fin(p, g, dinv, b, w):
    return pl.pallas_call(
        _fin_body,
        grid=(_G,),
        in_specs=[
            pl.BlockSpec((2, _R, D), lambda i: (0, i, 0)),
            pl.BlockSpec((_R, D), lambda i: (i, 0)),
            pl.BlockSpec((_R, 1), lambda i: (i, 0)),
            pl.BlockSpec((1, OUTP), lambda i: (0, 0)),
            pl.BlockSpec((D, OUTP), lambda i: (0, 0)),
        ],
        out_specs=pl.BlockSpec((_R, OUTP), lambda i: (i, 0)),
        out_shape=jax.ShapeDtypeStruct((N, OUTP), jnp.float32),
    )(p, g, dinv, b, w)


@jax.jit
def kernel(x, edge_index, W0, b0, W1, b1, W2, b2, W3, b3):
    pi = jnp.arange(EP - E, dtype=jnp.int32)
    pad = jnp.stack([pi % N, N + pi % (NP - N)])
    edges_r = jnp.concatenate([edge_index, pad], axis=1).reshape(
        2, NW, SEG, NCHUNK, CHUNK
    )
    zeros_full = jnp.zeros((NP, D), jnp.float32)
    ones = jnp.ones((CHUNK, D), jnp.float32)
    w3p = jnp.zeros((D, OUTP), jnp.float32).at[:, : W3.shape[1]].set(W3)
    b3p = jnp.zeros((1, OUTP), jnp.float32).at[0, : b3.shape[0]].set(b3)

    degp = _sc_deg(edges_r, zeros_full, ones)
    dinv, ht = _tc_prep(degp, x, W0)

    for b, w in ((b0, W1), (b1, W2)):
        p = _sc_scatter_full(ht, edges_r, zeros_full)
        ht = _tc_mid(p, ht, dinv, b.reshape(1, D), w)

    p = _sc_scatter_full(ht, edges_r, zeros_full)
    g = _tc_g(p, ht, dinv, b2.reshape(1, D))

    p = _sc_scatter_full(g, edges_r, zeros_full)
    out = _tc_fin(p, g, dinv, b3p, w3p)
    return out[:, :4]
